# Initial kernel scaffold; baseline (speedup 1.0000x reference)
import functools

import jax
import jax.numpy as jnp
from jax import lax
from jax.experimental import pallas as pl
from jax.experimental.pallas import tpu as pltpu
from jax.experimental.pallas import tpu_sc as plsc

NC = 2
NS = 16
NW = NC * NS
CHUNK = 128
NBUF = 2

_INV_PIO2 = 0.6366197723675814
_PIO2_A = 1.5703125
_PIO2_B = 4.837512969970703125e-4
_PIO2_C = 7.549789948768648e-8
_S3 = -1.6666654611e-1
_S5 = 8.3321608736e-3
_S7 = -1.9515295891e-4
_C2 = -0.5
_C4 = 4.166664568298827e-2
_C6 = -1.388731625493765e-3
_C8 = 2.443315711809948e-5


def _sincos(x):
    t = x * _INV_PIO2
    half = jnp.where(t >= 0.0, jnp.float32(0.5), jnp.float32(-0.5))
    ki = (t + half).astype(jnp.int32)
    kf = ki.astype(jnp.float32)
    r = x - kf * _PIO2_A
    r = r - kf * _PIO2_B
    r = r - kf * _PIO2_C
    r2 = r * r
    ps = ((_S7 * r2 + _S5) * r2 + _S3) * r2 * r + r
    pc = ((_C8 * r2 + _C6) * r2 + _C4) * (r2 * r2) + _C2 * r2 + 1.0
    swap = (ki & 1) == 1
    sv = jnp.where(swap, pc, ps)
    cv = jnp.where(swap, ps, pc)
    ssign = (ki & 2) << 30
    csign = ((ki + 1) & 2) << 30
    sinv = plsc.bitcast(plsc.bitcast(sv, jnp.int32) ^ ssign, jnp.float32)
    cosv = plsc.bitcast(plsc.bitcast(cv, jnp.int32) ^ csign, jnp.float32)
    return sinv, cosv


def _make_kernel(n_total, seq_len, dim):
    assert dim == 16
    nwk = n_total // NW
    nch = nwk // CHUNK
    assert n_total == NW * nch * CHUNK and nch % NBUF == 0

    mesh = plsc.VectorSubcoreMesh(
        core_axis_name="c", subcore_axis_name="s",
        num_cores=NC, num_subcores=NS)

    @functools.partial(
        pl.kernel,
        out_type=jax.ShapeDtypeStruct((n_total, 2 * dim), jnp.float32),
        mesh=mesh,
        scratch_types=[
            pltpu.VMEM((nch, CHUNK), jnp.int32),
            pltpu.VMEM((NBUF, CHUNK, dim), jnp.float32),
            pltpu.VMEM((NBUF, CHUNK, dim), jnp.float32),
            pltpu.VMEM((NBUF, CHUNK, 2 * dim), jnp.float32),
            pltpu.SemaphoreType.DMA,
            pltpu.SemaphoreType.DMA,
        ],
    )
    def kern(words_hbm, amp_hbm, freq_hbm, out_hbm,
             idx_v, amp_v, freq_v, out_v, sem0, sem1):
        wid = lax.axis_index("s") * NC + lax.axis_index("c")
        base = wid * nwk
        sems = (sem0, sem1)

        pltpu.sync_copy(words_hbm.at[wid], idx_v)

        def issue(c, b):
            pltpu.async_copy(amp_hbm.at[idx_v.at[c]], amp_v.at[b], sems[b])
            pltpu.async_copy(freq_hbm.at[idx_v.at[c]], freq_v.at[b], sems[b])

        def wait(b):
            pltpu.make_async_copy(
                amp_hbm.at[pl.ds(0, CHUNK)], amp_v.at[b], sems[b]).wait()
            pltpu.make_async_copy(
                freq_hbm.at[pl.ds(0, CHUNK)], freq_v.at[b], sems[b]).wait()

        for b in range(NBUF):
            issue(b, b)

        @pl.loop(0, nch, step=NBUF)
        def outer(c0):
            for b in range(NBUF):
                c = c0 + b
                gbase = base + c * CHUNK
                wait(b)

                @pl.loop(0, CHUNK, unroll=4)
                def row(j):
                    g = gbase + j
                    pos = ((g % seq_len) + 1).astype(jnp.float32)
                    fr = freq_v[b, j, :]
                    am = amp_v[b, j, :]
                    sinv, cosv = _sincos(fr * pos)
                    out_v[b, j, pl.ds(0, dim)] = am * cosv
                    out_v[b, j, pl.ds(dim, dim)] = am * sinv

                pltpu.sync_copy(out_v.at[b], out_hbm.at[pl.ds(gbase, CHUNK)])

                @pl.when(c + NBUF < nch)
                def _():
                    issue(c + NBUF, b)

    return kern


def kernel(words, amplitudes, frequencies):
    bsz, seq_len = words.shape
    dim = amplitudes.shape[1]
    n_total = bsz * seq_len
    nwk = n_total // NW
    words_grp = words.astype(jnp.int32).reshape(NW, nwk // CHUNK, CHUNK)
    out = _make_kernel(n_total, seq_len, dim)(
        words_grp, amplitudes, frequencies)
    return out.reshape(bsz, seq_len, 2 * dim)


# speedup vs baseline: 2.2601x; 2.2601x over previous
import functools

import jax
import jax.numpy as jnp
from jax import lax
from jax.experimental import pallas as pl
from jax.experimental.pallas import tpu as pltpu
from jax.experimental.pallas import tpu_sc as plsc

NC = 2
NS = 16
NW = NC * NS
BB = 128
NBUF = 2

_INV_PIO2 = 0.6366197723675814
_PIO2_A = 1.5703125
_PIO2_B = 4.837512969970703125e-4
_PIO2_C = 7.549789948768648e-8
_S3 = -1.6666654611e-1
_S5 = 8.3321608736e-3
_S7 = -1.9515295891e-4
_C4 = 4.166664568298827e-2
_C6 = -1.388731625493765e-3
_C8 = 2.443315711809948e-5


def _sincos(x):
    t = x * _INV_PIO2
    half = jnp.where(t >= 0.0, jnp.float32(0.5), jnp.float32(-0.5))
    ki = (t + half).astype(jnp.int32)
    kf = ki.astype(jnp.float32)
    r = x - kf * _PIO2_A
    r = r - kf * _PIO2_B
    r = r - kf * _PIO2_C
    r2 = r * r
    ps = ((_S7 * r2 + _S5) * r2 + _S3) * r2 * r + r
    pc = ((_C8 * r2 + _C6) * r2 + _C4) * (r2 * r2) - 0.5 * r2 + 1.0
    swap = (ki & 1) == 1
    sv = jnp.where(swap, pc, ps)
    cv = jnp.where(swap, ps, pc)
    ssign = (ki & 2) << 30
    csign = ((ki + 1) & 2) << 30
    sinv = lax.bitcast_convert_type(
        lax.bitcast_convert_type(sv, jnp.int32) ^ ssign, jnp.float32)
    cosv = lax.bitcast_convert_type(
        lax.bitcast_convert_type(cv, jnp.int32) ^ csign, jnp.float32)
    return sinv, cosv


def _make_linearize(vocab, dim):
    bw = 128
    nb = 4
    nt = vocab // bw
    tail = vocab - nt * bw
    iters = -(-nt // NW)
    iters += (-iters) % nb
    flat = jax.ShapeDtypeStruct((vocab * dim,), jnp.float32)

    mesh = plsc.VectorSubcoreMesh(
        core_axis_name="c", subcore_axis_name="s",
        num_cores=NC, num_subcores=NS)

    @functools.partial(
        pl.kernel,
        out_type=(flat, flat),
        mesh=mesh,
        scratch_types=[
            pltpu.VMEM((nb, dim, bw + 1), jnp.float32),
            pltpu.VMEM((nb, bw * dim), jnp.float32),
        ] + [pltpu.SemaphoreType.DMA] * (2 * nb),
        compiler_params=pltpu.CompilerParams(
            use_tc_tiling_on_sc=True, needs_layout_passes=False),
    )
    def lin(ampT, atail, freqT, ftail, ampL, freqL,
            tile_v, row_v, *sems):
        wid = lax.axis_index("s") * NC + lax.axis_index("c")
        isems = sems[:nb]
        osems = sems[nb:]
        iota = lax.iota(jnp.int32, 16)

        for src, dst in ((ampT, ampL), (freqT, freqL)):
            def issue_in(i, b, src=src):
                g = i * NW + wid

                @pl.when(g < nt)
                def _():
                    off = pl.multiple_of(g * bw, 128)
                    pltpu.async_copy(src.at[:, pl.ds(off, bw)],
                                     tile_v.at[b, :, pl.ds(0, bw)], isems[b])

            def wait_in(b, src=src):
                pltpu.make_async_copy(
                    src.at[:, pl.ds(0, bw)],
                    tile_v.at[b, :, pl.ds(0, bw)], isems[b]).wait()

            def wait_out(b, dst=dst):
                pltpu.make_async_copy(
                    dst.at[pl.ds(0, bw * dim)], row_v.at[b], osems[b]).wait()

            for b in range(nb):
                issue_in(b, b)

            @pl.loop(0, iters, step=nb)
            def blocks(c0):
                for b in range(nb):
                    i = c0 + b
                    g = i * NW + wid

                    @pl.when((i >= nb) & ((i - nb) * NW + wid < nt))
                    def _(b=b):
                        wait_out(b)

                    @pl.when(g < nt)
                    def _(i=i, b=b, g=g):
                        wait_in(b)

                        @plsc.parallel_loop(0, bw, unroll=8)
                        def rowf(c):
                            vals = plsc.load_gather(
                                tile_v.at[b], [iota, jnp.broadcast_to(c, (16,))])
                            row_v[b, pl.ds(c * dim, dim)] = vals

                        pltpu.async_copy(
                            row_v.at[b],
                            dst.at[pl.ds(g * bw * dim, bw * dim)], osems[b])
                        issue_in(i + nb, b)

            for b in range(nb):
                g_last = (iters - nb + b) * NW + wid

                @pl.when(g_last < nt)
                def _(b=b):
                    wait_out(b)

        if tail:
            @pl.when(wid == 0)
            def _():
                pltpu.sync_copy(atail, row_v.at[0, pl.ds(0, tail * dim)])
                pltpu.sync_copy(row_v.at[0, pl.ds(0, tail * dim)],
                                ampL.at[pl.ds(nt * bw * dim, tail * dim)])

            @pl.when(wid == 1)
            def _():
                pltpu.sync_copy(ftail, row_v.at[0, pl.ds(0, tail * dim)])
                pltpu.sync_copy(row_v.at[0, pl.ds(0, tail * dim)],
                                freqL.at[pl.ds(nt * bw * dim, tail * dim)])

    return lin


def _make_lookup(bsz, seq_len, dim):
    assert dim == 16 and bsz == NW * BB and seq_len % NBUF == 0
    out5 = (seq_len, 2 * dim // 8, bsz // BB, 8, BB)

    mesh = plsc.VectorSubcoreMesh(
        core_axis_name="c", subcore_axis_name="s",
        num_cores=NC, num_subcores=NS)

    @functools.partial(
        pl.kernel,
        out_type=jax.ShapeDtypeStruct(out5, jnp.float32),
        mesh=mesh,
        scratch_types=[
            pltpu.VMEM((BB, seq_len + 1), jnp.int32),
            pltpu.VMEM((NBUF, BB), jnp.int32),
            pltpu.VMEM((NBUF, BB, dim), jnp.float32),
            pltpu.VMEM((NBUF, BB, dim), jnp.float32),
            pltpu.VMEM((NBUF, 2 * dim // 8, 8, BB + 1), jnp.float32),
            pltpu.SemaphoreType.DMA,
            pltpu.SemaphoreType.DMA,
            pltpu.SemaphoreType.DMA,
            pltpu.SemaphoreType.DMA,
        ],
        compiler_params=pltpu.CompilerParams(
            use_tc_tiling_on_sc=False, needs_layout_passes=False),
    )
    def kern(words_hbm, amp_hbm, freq_hbm, o5_hbm,
             wstage, idx_c, amp_v, freq_v, out_v,
             gsem0, gsem1, osem0, osem1):
        wid = lax.axis_index("s") * NC + lax.axis_index("c")
        gsems = (gsem0, gsem1)
        osems = (osem0, osem1)

        pltpu.sync_copy(words_hbm.at[pl.ds(wid * BB, BB)],
                        wstage.at[:, pl.ds(0, seq_len)])

        iota = lax.iota(jnp.int32, 16)
        tc_re = lax.shift_right_logical(iota, 3)
        tc_im = tc_re + 2
        pc_i = iota & 7

        def build_idx(l, b):
            for k in range(BB // 16):
                row = k * 16 + iota
                col = jnp.broadcast_to(l, (16,))
                vals = plsc.load_gather(wstage, [row, col])
                idx_c[b, pl.ds(k * 16, 16)] = vals

        def issue(l, b):
            build_idx(l, b)
            pltpu.async_copy(amp_hbm.at[idx_c.at[b]], amp_v.at[b], gsems[b])
            pltpu.async_copy(freq_hbm.at[idx_c.at[b]], freq_v.at[b], gsems[b])

        def wait_g(b):
            pltpu.make_async_copy(
                amp_hbm.at[pl.ds(0, BB)], amp_v.at[b], gsems[b]).wait()
            pltpu.make_async_copy(
                freq_hbm.at[pl.ds(0, BB)], freq_v.at[b], gsems[b]).wait()

        def wait_o(b):
            pltpu.make_async_copy(
                o5_hbm.at[0, :, 0],
                out_v.at[b, :, :, pl.ds(0, BB)], osems[b]).wait()

        for b in range(NBUF):
            issue(b, b)

        @pl.loop(0, seq_len, step=NBUF)
        def outer(l0):
            for b in range(NBUF):
                l = l0 + b
                wait_g(b)

                @pl.when(l >= NBUF)
                def _():
                    wait_o(b)

                pos = (l + 1).astype(jnp.float32)

                @plsc.parallel_loop(0, BB, unroll=8)
                def row(j):
                    fr = freq_v[b, j, :]
                    am = amp_v[b, j, :]
                    sinv, cosv = _sincos(fr * pos)
                    jb = jnp.broadcast_to(j, (16,))
                    plsc.store_scatter(
                        out_v.at[b], [tc_re, pc_i, jb], am * cosv)
                    plsc.store_scatter(
                        out_v.at[b], [tc_im, pc_i, jb], am * sinv)

                pltpu.async_copy(out_v.at[b, :, :, pl.ds(0, BB)],
                                 o5_hbm.at[l, :, wid], osems[b])

                @pl.when(l + NBUF < seq_len)
                def _():
                    issue(l + NBUF, b)

        for b in range(NBUF):
            wait_o(b)

    return kern


def kernel(words, amplitudes, frequencies):
    bsz, seq_len = words.shape
    vocab, dim = amplitudes.shape
    nt = vocab // 128
    atail = amplitudes[nt * 128:, :].reshape(-1)
    ftail = frequencies[nt * 128:, :].reshape(-1)
    amp_flat, freq_flat = _make_linearize(vocab, dim)(
        amplitudes.T, atail, frequencies.T, ftail)
    o5 = _make_lookup(bsz, seq_len, dim)(
        words.astype(jnp.int32),
        amp_flat.reshape(vocab, dim), freq_flat.reshape(vocab, dim))
    return o5.transpose((2, 4, 0, 1, 3)).reshape(bsz, seq_len, 2 * dim)


# speedup vs baseline: 2.3196x; 1.0263x over previous
import functools

import jax
import jax.numpy as jnp
from jax import lax
from jax.experimental import pallas as pl
from jax.experimental.pallas import tpu as pltpu
from jax.experimental.pallas import tpu_sc as plsc

NC = 2
NS = 16
NW = NC * NS
BB = 128
NBUF = 2

_INV_PIO2 = 0.6366197723675814
_PIO2_A = 1.5703125
_PIO2_B = 4.837512969970703125e-4
_PIO2_C = 7.549789948768648e-8
_S3 = -1.6666654611e-1
_S5 = 8.3321608736e-3
_S7 = -1.9515295891e-4
_C4 = 4.166664568298827e-2
_C6 = -1.388731625493765e-3
_C8 = 2.443315711809948e-5


def _sincos(x):
    t = x * _INV_PIO2
    half = jnp.where(t >= 0.0, jnp.float32(0.5), jnp.float32(-0.5))
    ki = (t + half).astype(jnp.int32)
    kf = ki.astype(jnp.float32)
    r = x - kf * _PIO2_A
    r = r - kf * _PIO2_B
    r = r - kf * _PIO2_C
    r2 = r * r
    ps = ((_S7 * r2 + _S5) * r2 + _S3) * r2 * r + r
    pc = ((_C8 * r2 + _C6) * r2 + _C4) * (r2 * r2) - 0.5 * r2 + 1.0
    swap = (ki & 1) == 1
    sv = jnp.where(swap, pc, ps)
    cv = jnp.where(swap, ps, pc)
    ssign = (ki & 2) << 30
    csign = ((ki + 1) & 2) << 30
    sinv = lax.bitcast_convert_type(
        lax.bitcast_convert_type(sv, jnp.int32) ^ ssign, jnp.float32)
    cosv = lax.bitcast_convert_type(
        lax.bitcast_convert_type(cv, jnp.int32) ^ csign, jnp.float32)
    return sinv, cosv


def _make_linearize(vocab, dim):
    bw = 256
    nb = 4
    nt = vocab // bw
    tail = vocab - nt * bw
    iters = -(-nt // NW)
    iters += (-iters) % nb
    flat = jax.ShapeDtypeStruct((vocab * dim,), jnp.float32)

    mesh = plsc.VectorSubcoreMesh(
        core_axis_name="c", subcore_axis_name="s",
        num_cores=NC, num_subcores=NS)

    @functools.partial(
        pl.kernel,
        out_type=(flat, flat),
        mesh=mesh,
        scratch_types=[
            pltpu.VMEM((nb, dim, bw // 2 + 1), jnp.float32),
            pltpu.VMEM((nb, dim, bw // 2), jnp.float32),
            pltpu.VMEM((nb, bw * dim), jnp.float32),
            pltpu.VMEM((nb, bw // 2, dim + 1), jnp.float32),
        ] + [pltpu.SemaphoreType.DMA] * (2 * nb),
        compiler_params=pltpu.CompilerParams(
            use_tc_tiling_on_sc=True, needs_layout_passes=False),
    )
    def lin(ampT, atail, freqT, ftail, ampL, freqL,
            tile_a, tile_b, row_v, row_w, *sems):
        wid = lax.axis_index("s") * NC + lax.axis_index("c")
        isems = sems[:nb]
        osems = sems[nb:]
        iota = lax.iota(jnp.int32, 16)

        for src, dst in ((ampT, ampL), (freqT, freqL)):
            def issue_in(i, b, src=src):
                g = i * NW + wid

                @pl.when(g < nt)
                def _():
                    off = pl.multiple_of(g * bw, 128)
                    pltpu.async_copy(src.at[:, pl.ds(off, bw // 2)],
                                     tile_a.at[b, :, pl.ds(0, bw // 2)],
                                     isems[b])
                    off2 = pl.multiple_of(g * bw + bw // 2, 64)
                    pltpu.async_copy(src.at[:, pl.ds(off2, bw // 2)],
                                     tile_b.at[b], isems[b])

            def wait_in(b, src=src):
                pltpu.make_async_copy(
                    src.at[:, pl.ds(0, bw // 2)],
                    tile_a.at[b, :, pl.ds(0, bw // 2)], isems[b]).wait()
                pltpu.make_async_copy(
                    src.at[:, pl.ds(0, bw // 2)], tile_b.at[b],
                    isems[b]).wait()

            def wait_out(b, dst=dst):
                pltpu.make_async_copy(
                    dst.at[pl.ds(0, bw * dim)], row_v.at[b], osems[b]).wait()

            for b in range(nb):
                issue_in(b, b)

            @pl.loop(0, iters, step=nb)
            def blocks(c0):
                for b in range(nb):
                    i = c0 + b
                    g = i * NW + wid

                    @pl.when((i >= nb) & ((i - nb) * NW + wid < nt))
                    def _(b=b):
                        wait_out(b)

                    @pl.when(g < nt)
                    def _(i=i, b=b, g=g):
                        wait_in(b)

                        @plsc.parallel_loop(0, bw // 2, unroll=8)
                        def rowf(c):
                            vals = plsc.load_gather(
                                tile_a.at[b], [iota, jnp.broadcast_to(c, (16,))])
                            row_v[b, pl.ds(c * dim, dim)] = vals
                            d = c & (dim - 1)
                            c0 = c - d
                            svals = tile_b[b, d, pl.ds(c0, 16)]
                            plsc.store_scatter(
                                row_w.at[b],
                                [c0 + iota, jnp.broadcast_to(d, (16,))], svals)

                        @plsc.parallel_loop(0, bw // 2, unroll=8)
                        def bounce(r):
                            row_v[b, pl.ds((bw // 2 + r) * dim, dim)] = (
                                row_w[b, r, pl.ds(0, dim)])

                        pltpu.async_copy(
                            row_v.at[b],
                            dst.at[pl.ds(g * bw * dim, bw * dim)], osems[b])
                        issue_in(i + nb, b)

            for b in range(nb):
                g_last = (iters - nb + b) * NW + wid

                @pl.when(g_last < nt)
                def _(b=b):
                    wait_out(b)

        if tail:
            @pl.when(wid == 0)
            def _():
                pltpu.sync_copy(atail, row_v.at[0, pl.ds(0, tail * dim)])
                pltpu.sync_copy(row_v.at[0, pl.ds(0, tail * dim)],
                                ampL.at[pl.ds(nt * bw * dim, tail * dim)])

            @pl.when(wid == 1)
            def _():
                pltpu.sync_copy(ftail, row_v.at[0, pl.ds(0, tail * dim)])
                pltpu.sync_copy(row_v.at[0, pl.ds(0, tail * dim)],
                                freqL.at[pl.ds(nt * bw * dim, tail * dim)])

    return lin


def _make_lookup(bsz, seq_len, dim):
    assert dim == 16 and bsz == NW * BB and seq_len % NBUF == 0
    out5 = (seq_len, 2 * dim // 8, bsz // BB, 8, BB)

    mesh = plsc.VectorSubcoreMesh(
        core_axis_name="c", subcore_axis_name="s",
        num_cores=NC, num_subcores=NS)

    @functools.partial(
        pl.kernel,
        out_type=jax.ShapeDtypeStruct(out5, jnp.float32),
        mesh=mesh,
        scratch_types=[
            pltpu.VMEM((BB, seq_len + 1), jnp.int32),
            pltpu.VMEM((NBUF, BB), jnp.int32),
            pltpu.VMEM((NBUF, BB, dim), jnp.float32),
            pltpu.VMEM((NBUF, BB, dim), jnp.float32),
            pltpu.VMEM((NBUF, 2 * dim // 8, 8, BB + 1), jnp.float32),
            pltpu.SemaphoreType.DMA,
            pltpu.SemaphoreType.DMA,
            pltpu.SemaphoreType.DMA,
            pltpu.SemaphoreType.DMA,
        ],
        compiler_params=pltpu.CompilerParams(
            use_tc_tiling_on_sc=False, needs_layout_passes=False),
    )
    def kern(words_hbm, amp_hbm, freq_hbm, o5_hbm,
             wstage, idx_c, amp_v, freq_v, out_v,
             gsem0, gsem1, osem0, osem1):
        wid = lax.axis_index("s") * NC + lax.axis_index("c")
        gsems = (gsem0, gsem1)
        osems = (osem0, osem1)

        pltpu.sync_copy(words_hbm.at[pl.ds(wid * BB, BB)],
                        wstage.at[:, pl.ds(0, seq_len)])

        iota = lax.iota(jnp.int32, 16)
        tc_re = lax.shift_right_logical(iota, 3)
        tc_im = tc_re + 2
        pc_i = iota & 7

        def build_idx(l, b):
            for k in range(BB // 16):
                row = k * 16 + iota
                col = jnp.broadcast_to(l, (16,))
                vals = plsc.load_gather(wstage, [row, col])
                idx_c[b, pl.ds(k * 16, 16)] = vals

        def issue(l, b):
            build_idx(l, b)
            pltpu.async_copy(amp_hbm.at[idx_c.at[b]], amp_v.at[b], gsems[b])
            pltpu.async_copy(freq_hbm.at[idx_c.at[b]], freq_v.at[b], gsems[b])

        def wait_g(b):
            pltpu.make_async_copy(
                amp_hbm.at[pl.ds(0, BB)], amp_v.at[b], gsems[b]).wait()
            pltpu.make_async_copy(
                freq_hbm.at[pl.ds(0, BB)], freq_v.at[b], gsems[b]).wait()

        def wait_o(b):
            pltpu.make_async_copy(
                o5_hbm.at[0, :, 0],
                out_v.at[b, :, :, pl.ds(0, BB)], osems[b]).wait()

        for b in range(NBUF):
            issue(b, b)

        @pl.loop(0, seq_len, step=NBUF)
        def outer(l0):
            for b in range(NBUF):
                l = l0 + b
                wait_g(b)

                @pl.when(l >= NBUF)
                def _():
                    wait_o(b)

                pos = (l + 1).astype(jnp.float32)

                @plsc.parallel_loop(0, BB, unroll=8)
                def row(j):
                    fr = freq_v[b, j, :]
                    am = amp_v[b, j, :]
                    sinv, cosv = _sincos(fr * pos)
                    jb = jnp.broadcast_to(j, (16,))
                    plsc.store_scatter(
                        out_v.at[b], [tc_re, pc_i, jb], am * cosv)
                    plsc.store_scatter(
                        out_v.at[b], [tc_im, pc_i, jb], am * sinv)

                pltpu.async_copy(out_v.at[b, :, :, pl.ds(0, BB)],
                                 o5_hbm.at[l, :, wid], osems[b])

                @pl.when(l + NBUF < seq_len)
                def _():
                    issue(l + NBUF, b)

        for b in range(NBUF):
            wait_o(b)

    return kern


def kernel(words, amplitudes, frequencies):
    bsz, seq_len = words.shape
    vocab, dim = amplitudes.shape
    nt = vocab // 128
    atail = amplitudes[nt * 128:, :].reshape(-1)
    ftail = frequencies[nt * 128:, :].reshape(-1)
    amp_flat, freq_flat = _make_linearize(vocab, dim)(
        amplitudes.T, atail, frequencies.T, ftail)
    o5 = _make_lookup(bsz, seq_len, dim)(
        words.astype(jnp.int32),
        amp_flat.reshape(vocab, dim), freq_flat.reshape(vocab, dim))
    return o5.transpose((2, 4, 0, 1, 3)).reshape(bsz, seq_len, 2 * dim)
